# baseline (device time: 38054 ns/iter reference)
import jax
import jax.numpy as jnp
from jax import lax
from jax.experimental import pallas as pl
from jax.experimental.pallas import tpu as pltpu

N_DEV = 4
SQ = 256
QROWS = SQ // N_DEV
D = 1024
HQ = 8
HKV = 2
DH = 128
GQ = HQ // HKV
GD = GQ * DH
NCH = 2
SCALE = 0.08838834764831843


def kernel(x, Wq, Wo, K_ext, V_ext):
    skv = K_ext.shape[1]
    ch = skv // NCH

    def body(x_ref, wq_ref, wo_ref, k_ref, v_ref, out_ref,
             k_buf, v_buf, wo_buf, loc_stage, mlrow_stage, part_comm,
             ag_stage, ag_comm,
             k_sem, v_sem, wo_sem, rs_send, rs_recv, ag_send, ag_recv):
        my = lax.axis_index("i")
        left = (my + N_DEV - 1) % N_DEV
        right = (my + 1) % N_DEV
        diag = (my + 2) % N_DEV

        def kv_copy(g, c, slot):
            kc = pltpu.make_async_copy(
                k_ref.at[pl.ds(c * ch, ch), pl.ds(g * DH, DH)],
                k_buf.at[slot], k_sem.at[slot])
            vc = pltpu.make_async_copy(
                v_ref.at[pl.ds(c * ch, ch), pl.ds(g * DH, DH)],
                v_buf.at[slot], v_sem.at[slot])
            kc.start()
            vc.start()
            return kc, vc

        cp = {(0, c): kv_copy(0, c, c) for c in range(NCH)}
        wo_cp = pltpu.make_async_copy(wo_ref, wo_buf, wo_sem)
        wo_cp.start()

        barrier = pltpu.get_barrier_semaphore()
        for nbr in (left, right):
            pl.semaphore_signal(barrier, inc=1, device_id=(nbr,),
                                device_id_type=pl.DeviceIdType.MESH)
        pl.semaphore_wait(barrier, 2)

        xb = x_ref[0].astype(jnp.bfloat16)
        wq = wq_ref[:].astype(jnp.bfloat16)
        q = lax.dot_general(xb, wq, (((1,), (0,)), ((), ())),
                            preferred_element_type=jnp.float32)
        q = q.astype(jnp.bfloat16)

        dests = (right, left, diag)

        rdmas = []
        for g in range(HKV):
            m_g = [None] * GQ
            l_g = [None] * GQ
            o_g = [None] * GQ
            for c in range(NCH):
                kc, vc = cp[(g, c)]
                kc.wait()
                vc.wait()
                kg = k_buf[c].astype(jnp.bfloat16)
                vg = v_buf[c].astype(jnp.bfloat16)
                for hh in range(GQ):
                    h = g * GQ + hh
                    qh = q[:, h * DH:(h + 1) * DH]
                    s = lax.dot_general(
                        qh, kg, (((1,), (1,)), ((), ())),
                        preferred_element_type=jnp.float32) * SCALE
                    mc = jnp.max(s, axis=1, keepdims=True)
                    if c == 0:
                        p = jnp.exp(s - mc)
                        m_g[hh] = mc
                        l_g[hh] = jnp.sum(p, axis=1, keepdims=True)
                        o_g[hh] = lax.dot_general(
                            p.astype(jnp.bfloat16), vg,
                            (((1,), (0,)), ((), ())),
                            preferred_element_type=jnp.float32)
                    else:
                        m_new = jnp.maximum(m_g[hh], mc)
                        alpha = jnp.exp(m_g[hh] - m_new)
                        p = jnp.exp(s - m_new)
                        l_g[hh] = l_g[hh] * alpha + jnp.sum(
                            p, axis=1, keepdims=True)
                        o_g[hh] = o_g[hh] * alpha + lax.dot_general(
                            p.astype(jnp.bfloat16), vg,
                            (((1,), (0,)), ((), ())),
                            preferred_element_type=jnp.float32)
                        m_g[hh] = m_new
                if g + 1 < HKV:
                    cp[(g + 1, c)] = kv_copy(g + 1, c, c)

            ml_row = jnp.concatenate(m_g + l_g, axis=1)
            mlrow_stage[g] = ml_row
            loc_stage[g] = jnp.concatenate(
                o_g + [ml_row], axis=1).astype(jnp.bfloat16)

            for dest, dev in enumerate(dests):
                idx = dest * HKV + g
                r = pltpu.make_async_remote_copy(
                    src_ref=loc_stage.at[g, pl.ds(dev * QROWS, QROWS), :],
                    dst_ref=part_comm.at[dest, g],
                    send_sem=rs_send.at[idx], recv_sem=rs_recv.at[idx],
                    device_id=(dev,), device_id_type=pl.DeviceIdType.MESH)
                r.start()
                rdmas.append(r)

        wo_cp.wait()
        wo = wo_buf[:].astype(jnp.bfloat16)
        out = None
        for g in range(HKV):
            ml_q = mlrow_stage[g, pl.ds(my * QROWS, QROWS), :]
            m_acc = ml_q[:, 0:GQ]
            l_acc = ml_q[:, GQ:2 * GQ]
            o_q = loc_stage[g, pl.ds(my * QROWS, QROWS), :].astype(jnp.float32)
            o_acc = [o_q[:, hh * DH:(hh + 1) * DH] for hh in range(GQ)]
            for dest in range(3):
                r = rdmas[g * 3 + dest]
                r.wait_recv()
                blk = part_comm[dest, g].astype(jnp.float32)
                m_r = blk[:, GD:GD + GQ]
                l_r = blk[:, GD + GQ:GD + 2 * GQ]
                m_new = jnp.maximum(m_acc, m_r)
                a_o = jnp.exp(m_acc - m_new)
                a_r = jnp.exp(m_r - m_new)
                l_acc = l_acc * a_o + l_r * a_r
                o_acc = [o_acc[hh] * a_o[:, hh:hh + 1]
                         + blk[:, hh * DH:(hh + 1) * DH] * a_r[:, hh:hh + 1]
                         for hh in range(GQ)]
                m_acc = m_new
            attn_g = jnp.concatenate(
                [o_acc[hh] / l_acc[:, hh:hh + 1] for hh in range(GQ)], axis=1)
            part = lax.dot_general(attn_g.astype(jnp.bfloat16),
                                   wo[g * GD:(g + 1) * GD, :],
                                   (((1,), (0,)), ((), ())),
                                   preferred_element_type=jnp.float32)
            out = part if out is None else out + part

        out_ref[0, pl.ds(my * QROWS, QROWS), :] = out
        ag_stage[:] = out.astype(jnp.bfloat16)
        ag_rdmas = []
        for dest, dev in enumerate(dests):
            r = pltpu.make_async_remote_copy(
                src_ref=ag_stage, dst_ref=ag_comm.at[dest],
                send_sem=ag_send.at[dest], recv_sem=ag_recv.at[dest],
                device_id=(dev,), device_id_type=pl.DeviceIdType.MESH)
            r.start()
            ag_rdmas.append(r)

        for dest, origin in enumerate((left, right, diag)):
            ag_rdmas[dest].wait_recv()
            out_ref[0, pl.ds(origin * QROWS, QROWS), :] = (
                ag_comm[dest].astype(jnp.float32))

        for r in rdmas:
            r.wait_send()
        for r in ag_rdmas:
            r.wait_send()

    K2 = K_ext.reshape(skv, HKV * DH)
    V2 = V_ext.reshape(skv, HKV * DH)

    return pl.pallas_call(
        body,
        out_shape=jax.ShapeDtypeStruct((1, SQ, D), jnp.float32),
        in_specs=[
            pl.BlockSpec(memory_space=pltpu.VMEM),
            pl.BlockSpec(memory_space=pltpu.VMEM),
            pl.BlockSpec(memory_space=pltpu.MemorySpace.HBM),
            pl.BlockSpec(memory_space=pltpu.MemorySpace.HBM),
            pl.BlockSpec(memory_space=pltpu.MemorySpace.HBM),
        ],
        out_specs=pl.BlockSpec(memory_space=pltpu.VMEM),
        scratch_shapes=[
            pltpu.VMEM((NCH, ch, DH), jnp.float32),
            pltpu.VMEM((NCH, ch, DH), jnp.float32),
            pltpu.VMEM((D, D), jnp.float32),
            pltpu.VMEM((HKV, SQ, GD + 2 * GQ), jnp.bfloat16),
            pltpu.VMEM((HKV, SQ, 2 * GQ), jnp.float32),
            pltpu.VMEM((3, HKV, QROWS, GD + 2 * GQ), jnp.bfloat16),
            pltpu.VMEM((QROWS, D), jnp.bfloat16),
            pltpu.VMEM((3, QROWS, D), jnp.bfloat16),
            pltpu.SemaphoreType.DMA((NCH,)),
            pltpu.SemaphoreType.DMA((NCH,)),
            pltpu.SemaphoreType.DMA,
            pltpu.SemaphoreType.DMA((6,)),
            pltpu.SemaphoreType.DMA((6,)),
            pltpu.SemaphoreType.DMA((3,)),
            pltpu.SemaphoreType.DMA((3,)),
        ],
        compiler_params=pltpu.CompilerParams(collective_id=0),
    )(x, Wq, Wo, K2, V2)


# device time: 35396 ns/iter; 1.0751x vs baseline; 1.0751x over previous
import jax
import jax.numpy as jnp
from jax import lax
from jax.experimental import pallas as pl
from jax.experimental.pallas import tpu as pltpu

N_DEV = 4
SQ = 256
QROWS = SQ // N_DEV
D = 1024
HQ = 8
HKV = 2
DH = 128
GQ = HQ // HKV
GD = GQ * DH
SCALE = 0.08838834764831843


def kernel(x, Wq, Wo, K_ext, V_ext):
    skv = K_ext.shape[1]

    def body(x_ref, wq_ref, wo_ref, k_ref, v_ref, out_ref,
             loc_stage, part_comm, mlrow_stage,
             ag_stage, ag_comm,
             rs_send, rs_recv, ag_send, ag_recv):
        my = lax.axis_index("i")
        left = (my + N_DEV - 1) % N_DEV
        right = (my + 1) % N_DEV
        diag = (my + 2) % N_DEV

        barrier = pltpu.get_barrier_semaphore()
        for nbr in (left, right):
            pl.semaphore_signal(barrier, inc=1, device_id=(nbr,),
                                device_id_type=pl.DeviceIdType.MESH)
        pl.semaphore_wait(barrier, 2)

        xb = x_ref[0].astype(jnp.bfloat16)
        wq = wq_ref[:].astype(jnp.bfloat16)
        q = lax.dot_general(xb, wq, (((1,), (0,)), ((), ())),
                            preferred_element_type=jnp.float32)
        q = q.astype(jnp.bfloat16)

        dests = (right, left, diag)

        o_loc, m_loc, l_loc, rdmas = {}, {}, {}, []
        for g in range(HKV):
            kg = k_ref[:, g * DH:(g + 1) * DH].astype(jnp.bfloat16)
            vg = v_ref[:, g * DH:(g + 1) * DH].astype(jnp.bfloat16)
            o_g, m_g, l_g = [], [], []
            for hh in range(GQ):
                h = g * GQ + hh
                qh = q[:, h * DH:(h + 1) * DH]
                s = lax.dot_general(qh, kg, (((1,), (1,)), ((), ())),
                                    preferred_element_type=jnp.float32) * SCALE
                mh = jnp.max(s, axis=1, keepdims=True)
                p = jnp.exp(s - mh)
                lh = jnp.sum(p, axis=1, keepdims=True)
                oh = lax.dot_general(p.astype(jnp.bfloat16), vg,
                                     (((1,), (0,)), ((), ())),
                                     preferred_element_type=jnp.float32)
                o_g.append(oh)
                m_g.append(mh)
                l_g.append(lh)

            m_loc[g] = jnp.concatenate(m_g, axis=1)
            l_loc[g] = jnp.concatenate(l_g, axis=1)
            o_loc[g] = o_g

            ml_row = jnp.concatenate([m_loc[g], l_loc[g]], axis=1)
            mlrow_stage[g] = ml_row
            loc_stage[g] = jnp.concatenate(
                o_g + [ml_row], axis=1).astype(jnp.bfloat16)

            for dest, dev in enumerate(dests):
                idx = dest * HKV + g
                o_r = pltpu.make_async_remote_copy(
                    src_ref=loc_stage.at[g, pl.ds(dev * QROWS, QROWS), :],
                    dst_ref=part_comm.at[dest, g],
                    send_sem=rs_send.at[idx], recv_sem=rs_recv.at[idx],
                    device_id=(dev,), device_id_type=pl.DeviceIdType.MESH)
                o_r.start()
                rdmas.append(o_r)

        attn = []
        wo = wo_ref[:].astype(jnp.bfloat16)
        for g in range(HKV):
            ml_q = mlrow_stage[g, pl.ds(my * QROWS, QROWS), :]
            m_acc = ml_q[:, 0:GQ]
            l_acc = ml_q[:, GQ:2 * GQ]
            o_q = loc_stage[g, pl.ds(my * QROWS, QROWS), :].astype(jnp.float32)
            o_acc = [o_q[:, hh * DH:(hh + 1) * DH] for hh in range(GQ)]
            for dest in range(3):
                o_r = rdmas[g * 3 + dest]
                o_r.wait_recv()
                blk = part_comm[dest, g].astype(jnp.float32)
                ml_t = blk[:, HQ * DH // HKV:]
                m_r = ml_t[:, 0:GQ]
                l_r = ml_t[:, GQ:2 * GQ]
                m_new = jnp.maximum(m_acc, m_r)
                a_o = jnp.exp(m_acc - m_new)
                a_r = jnp.exp(m_r - m_new)
                l_acc = l_acc * a_o + l_r * a_r
                o_part = blk
                o_acc = [o_acc[hh] * a_o[:, hh:hh + 1]
                         + o_part[:, hh * DH:(hh + 1) * DH] * a_r[:, hh:hh + 1]
                         for hh in range(GQ)]
                m_acc = m_new
            attn_g = jnp.concatenate(
                [o_acc[hh] / l_acc[:, hh:hh + 1] for hh in range(GQ)], axis=1)
            attn.append(attn_g.astype(jnp.bfloat16))

        ag_rdmas = []
        for half in range(2):
            outh = None
            for g in range(HKV):
                p = lax.dot_general(
                    attn[g], wo[g * GD:(g + 1) * GD,
                                half * GD:(half + 1) * GD],
                    (((1,), (0,)), ((), ())),
                    preferred_element_type=jnp.float32)
                outh = p if outh is None else outh + p
            out_ref[0, pl.ds(my * QROWS, QROWS),
                    half * GD:(half + 1) * GD] = outh
            ag_stage[:, half * GD:(half + 1) * GD] = outh.astype(jnp.bfloat16)
            for dest, dev in enumerate(dests):
                idx = dest * 2 + half
                r = pltpu.make_async_remote_copy(
                    src_ref=ag_stage.at[:, pl.ds(half * GD, GD)],
                    dst_ref=ag_comm.at[dest, :, pl.ds(half * GD, GD)],
                    send_sem=ag_send.at[idx], recv_sem=ag_recv.at[idx],
                    device_id=(dev,), device_id_type=pl.DeviceIdType.MESH)
                r.start()
                ag_rdmas.append(r)

        for dest, origin in enumerate((left, right, diag)):
            for half in range(2):
                ag_rdmas[half * 3 + dest].wait_recv()
                out_ref[0, pl.ds(origin * QROWS, QROWS),
                        half * GD:(half + 1) * GD] = (
                    ag_comm[dest, :, half * GD:(half + 1) * GD]
                    .astype(jnp.float32))

        for o_r in rdmas:
            o_r.wait_send()
        for r in ag_rdmas:
            r.wait_send()

    K2 = K_ext.reshape(skv, HKV * DH)
    V2 = V_ext.reshape(skv, HKV * DH)

    return pl.pallas_call(
        body,
        out_shape=jax.ShapeDtypeStruct((1, SQ, D), jnp.float32),
        in_specs=[pl.BlockSpec(memory_space=pltpu.VMEM)] * 5,
        out_specs=pl.BlockSpec(memory_space=pltpu.VMEM),
        scratch_shapes=[
            pltpu.VMEM((HKV, SQ, GD + 2 * GQ), jnp.bfloat16),
            pltpu.VMEM((3, HKV, QROWS, GD + 2 * GQ), jnp.bfloat16),
            pltpu.VMEM((HKV, SQ, 2 * GQ), jnp.float32),
            pltpu.VMEM((QROWS, D), jnp.bfloat16),
            pltpu.VMEM((3, QROWS, D), jnp.bfloat16),
            pltpu.SemaphoreType.DMA((6,)),
            pltpu.SemaphoreType.DMA((6,)),
            pltpu.SemaphoreType.DMA((6,)),
            pltpu.SemaphoreType.DMA((6,)),
        ],
        compiler_params=pltpu.CompilerParams(collective_id=0),
    )(x, Wq, Wo, K2, V2)


# device time: 35091 ns/iter; 1.0844x vs baseline; 1.0087x over previous
import jax
import jax.numpy as jnp
from jax import lax
from jax.experimental import pallas as pl
from jax.experimental.pallas import tpu as pltpu

N_DEV = 4
SQ = 256
QROWS = SQ // N_DEV
D = 1024
HQ = 8
HKV = 2
DH = 128
GQ = HQ // HKV
GD = GQ * DH
SCALE = 0.08838834764831843


def kernel(x, Wq, Wo, K_ext, V_ext):
    skv = K_ext.shape[1]

    def body(x_ref, wq_ref, wo_ref, k_ref, v_ref, out_ref,
             loc_stage, part_comm, mlrow_stage,
             ag_stage, ag_comm,
             rs_send, rs_recv, ag_send, ag_recv):
        my = lax.axis_index("i")
        left = (my + N_DEV - 1) % N_DEV
        right = (my + 1) % N_DEV
        diag = (my + 2) % N_DEV

        barrier = pltpu.get_barrier_semaphore()
        for nbr in (left, right):
            pl.semaphore_signal(barrier, inc=1, device_id=(nbr,),
                                device_id_type=pl.DeviceIdType.MESH)
        pl.semaphore_wait(barrier, 2)

        xb = x_ref[0].astype(jnp.bfloat16)
        wq = wq_ref[:].astype(jnp.bfloat16)
        q = lax.dot_general(xb, wq, (((1,), (0,)), ((), ())),
                            preferred_element_type=jnp.float32)
        q = q.astype(jnp.bfloat16)

        dests = (right, left, diag)

        o_loc, m_loc, l_loc, rdmas = {}, {}, {}, []
        for g in range(HKV):
            kg = k_ref[:, g * DH:(g + 1) * DH].astype(jnp.bfloat16)
            vg = v_ref[:, g * DH:(g + 1) * DH].astype(jnp.bfloat16)
            o_g, m_g, l_g = [], [], []
            for hh in range(GQ):
                h = g * GQ + hh
                qh = q[:, h * DH:(h + 1) * DH]
                s = lax.dot_general(qh, kg, (((1,), (1,)), ((), ())),
                                    preferred_element_type=jnp.float32) * SCALE
                mh = jnp.max(s, axis=1, keepdims=True)
                p = jnp.exp((s - mh).astype(jnp.bfloat16))
                lh = jnp.sum(p, axis=1, keepdims=True,
                             dtype=jnp.float32)
                oh = lax.dot_general(p, vg,
                                     (((1,), (0,)), ((), ())),
                                     preferred_element_type=jnp.float32)
                o_g.append(oh)
                m_g.append(mh)
                l_g.append(lh)

            m_loc[g] = jnp.concatenate(m_g, axis=1)
            l_loc[g] = jnp.concatenate(l_g, axis=1)
            o_loc[g] = o_g

            ml_row = jnp.concatenate([m_loc[g], l_loc[g]], axis=1)
            mlrow_stage[g] = ml_row
            loc_stage[g] = jnp.concatenate(
                o_g + [ml_row], axis=1).astype(jnp.bfloat16)

            for dest, dev in enumerate(dests):
                idx = dest * HKV + g
                o_r = pltpu.make_async_remote_copy(
                    src_ref=loc_stage.at[g, pl.ds(dev * QROWS, QROWS), :],
                    dst_ref=part_comm.at[dest, g],
                    send_sem=rs_send.at[idx], recv_sem=rs_recv.at[idx],
                    device_id=(dev,), device_id_type=pl.DeviceIdType.MESH)
                o_r.start()
                rdmas.append(o_r)

        attn = []
        wo = wo_ref[:].astype(jnp.bfloat16)
        for g in range(HKV):
            ml_q = mlrow_stage[g, pl.ds(my * QROWS, QROWS), :]
            m_acc = ml_q[:, 0:GQ]
            l_acc = ml_q[:, GQ:2 * GQ]
            o_q = loc_stage[g, pl.ds(my * QROWS, QROWS), :].astype(jnp.float32)
            o_acc = [o_q[:, hh * DH:(hh + 1) * DH] for hh in range(GQ)]
            for dest in range(3):
                o_r = rdmas[g * 3 + dest]
                o_r.wait_recv()
                blk = part_comm[dest, g].astype(jnp.float32)
                ml_t = blk[:, HQ * DH // HKV:]
                m_r = ml_t[:, 0:GQ]
                l_r = ml_t[:, GQ:2 * GQ]
                m_new = jnp.maximum(m_acc, m_r)
                a_o = jnp.exp(m_acc - m_new)
                a_r = jnp.exp(m_r - m_new)
                l_acc = l_acc * a_o + l_r * a_r
                o_part = blk
                o_acc = [o_acc[hh] * a_o[:, hh:hh + 1]
                         + o_part[:, hh * DH:(hh + 1) * DH] * a_r[:, hh:hh + 1]
                         for hh in range(GQ)]
                m_acc = m_new
            attn_g = jnp.concatenate(
                [o_acc[hh] / l_acc[:, hh:hh + 1] for hh in range(GQ)], axis=1)
            attn.append(attn_g.astype(jnp.bfloat16))

        ag_rdmas = []
        for half in range(2):
            outh = None
            for g in range(HKV):
                p = lax.dot_general(
                    attn[g], wo[g * GD:(g + 1) * GD,
                                half * GD:(half + 1) * GD],
                    (((1,), (0,)), ((), ())),
                    preferred_element_type=jnp.float32)
                outh = p if outh is None else outh + p
            out_ref[0, pl.ds(my * QROWS, QROWS),
                    half * GD:(half + 1) * GD] = outh
            ag_stage[:, half * GD:(half + 1) * GD] = outh.astype(jnp.bfloat16)
            for dest, dev in enumerate(dests):
                idx = dest * 2 + half
                r = pltpu.make_async_remote_copy(
                    src_ref=ag_stage.at[:, pl.ds(half * GD, GD)],
                    dst_ref=ag_comm.at[dest, :, pl.ds(half * GD, GD)],
                    send_sem=ag_send.at[idx], recv_sem=ag_recv.at[idx],
                    device_id=(dev,), device_id_type=pl.DeviceIdType.MESH)
                r.start()
                ag_rdmas.append(r)

        for dest, origin in enumerate((left, right, diag)):
            for half in range(2):
                ag_rdmas[half * 3 + dest].wait_recv()
                out_ref[0, pl.ds(origin * QROWS, QROWS),
                        half * GD:(half + 1) * GD] = (
                    ag_comm[dest, :, half * GD:(half + 1) * GD]
                    .astype(jnp.float32))

        for o_r in rdmas:
            o_r.wait_send()
        for r in ag_rdmas:
            r.wait_send()

    K2 = K_ext.reshape(skv, HKV * DH)
    V2 = V_ext.reshape(skv, HKV * DH)

    return pl.pallas_call(
        body,
        out_shape=jax.ShapeDtypeStruct((1, SQ, D), jnp.float32),
        in_specs=[pl.BlockSpec(memory_space=pltpu.VMEM)] * 5,
        out_specs=pl.BlockSpec(memory_space=pltpu.VMEM),
        scratch_shapes=[
            pltpu.VMEM((HKV, SQ, GD + 2 * GQ), jnp.bfloat16),
            pltpu.VMEM((3, HKV, QROWS, GD + 2 * GQ), jnp.bfloat16),
            pltpu.VMEM((HKV, SQ, 2 * GQ), jnp.float32),
            pltpu.VMEM((QROWS, D), jnp.bfloat16),
            pltpu.VMEM((3, QROWS, D), jnp.bfloat16),
            pltpu.SemaphoreType.DMA((6,)),
            pltpu.SemaphoreType.DMA((6,)),
            pltpu.SemaphoreType.DMA((6,)),
            pltpu.SemaphoreType.DMA((6,)),
        ],
        compiler_params=pltpu.CompilerParams(collective_id=0),
    )(x, Wq, Wo, K2, V2)
